# baseline (speedup 1.0000x reference)
import functools

import jax
import jax.numpy as jnp
from jax import lax
from jax.experimental import pallas as pl
from jax.experimental.pallas import tpu as pltpu
from jax.experimental.pallas import tpu_sc as plsc

N = 10000
E = 320000
EN = E + N
F_IN = 128
C = 16
H = 8
H2 = 2 * H
D_E = 4
SLOPE = 0.2

NC = 2
NS = 16
NW = NC * NS
HB = 128
PER_W = 10752
E_PAD = NW * PER_W
B1 = 1536
NQ1 = B1 // HB
NCH1 = PER_W // B1
B2 = 256
NQ2 = B2 // HB
NCH2 = PER_W // B2
EA_ROWS = E + B1
RPS = 640
RPS_LAST = N - 15 * RPS

_mesh = plsc.VectorSubcoreMesh(core_axis_name="c", subcore_axis_name="s")
_sc_params = pltpu.CompilerParams(use_tc_tiling_on_sc=False)


def _copy_rows(s, src, dst):
    @pl.when(s < NS - 1)
    def _():
        off = pl.multiple_of(s * RPS, 8)
        pltpu.sync_copy(src.at[pl.ds(off, RPS)], dst.at[pl.ds(off, RPS)])

    @pl.when(s == NS - 1)
    def _():
        pltpu.sync_copy(src.at[pl.ds(15 * RPS, RPS_LAST)],
                        dst.at[pl.ds(15 * RPS, RPS_LAST)])


def _proj_body(x_ref, w_ref, as_ref, ad_ref, h_ref, asrc_ref, adst_ref):
    h = jnp.dot(x_ref[...], w_ref[...], preferred_element_type=jnp.float32)
    h_ref[...] = h
    asrc_ref[...] = jnp.dot(h, as_ref[...], preferred_element_type=jnp.float32)
    adst_ref[...] = jnp.dot(h, ad_ref[...], preferred_element_type=jnp.float32)


def _project(x, W, Asrc, Adst):
    blk = 1000
    return pl.pallas_call(
        _proj_body,
        grid=(N // blk,),
        in_specs=[pl.BlockSpec((blk, F_IN), lambda i: (i, 0)),
                  pl.BlockSpec((F_IN, F_IN), lambda i: (0, 0)),
                  pl.BlockSpec((F_IN, H2), lambda i: (0, 0)),
                  pl.BlockSpec((F_IN, H2), lambda i: (0, 0))],
        out_specs=[pl.BlockSpec((blk, F_IN), lambda i: (i, 0)),
                   pl.BlockSpec((blk, H2), lambda i: (i, 0)),
                   pl.BlockSpec((blk, H2), lambda i: (i, 0))],
        out_shape=[jax.ShapeDtypeStruct((N, F_IN), jnp.float32),
                   jax.ShapeDtypeStruct((N, H2), jnp.float32),
                   jax.ShapeDtypeStruct((N, H2), jnp.float32)],
    )(x, W, Asrc, Adst)


_EB = EA_ROWS // 64


def _ae_body(ea_ref, bm_ref, ae_ref, al_ref, acc_ref):
    i = pl.program_id(0)
    ae = jnp.dot(ea_ref[...], bm_ref[...], preferred_element_type=jnp.float32)
    ae_ref[...] = ae

    @pl.when(i == 0)
    def _():
        acc_ref[...] = jnp.zeros_like(acc_ref)

    acc_ref[...] += ae

    @pl.when(i == pl.num_programs(0) - 1)
    def _():
        s = jnp.sum(acc_ref[...], axis=0, keepdims=True) * (1.0 / E)
        al_ref[...] = jnp.broadcast_to(s, al_ref.shape)


def _ae(ea8, Bmat16):
    return pl.pallas_call(
        _ae_body,
        grid=(EA_ROWS // _EB,),
        in_specs=[pl.BlockSpec((_EB, H), lambda i: (i, 0)),
                  pl.BlockSpec((H, H2), lambda i: (0, 0))],
        out_specs=[pl.BlockSpec((_EB, H2), lambda i: (i, 0)),
                   pl.BlockSpec((8, H2), lambda i: (0, 0))],
        out_shape=[jax.ShapeDtypeStruct((EA_ROWS, H2), jnp.float32),
                   jax.ShapeDtypeStruct((8, H2), jnp.float32)],
        scratch_shapes=[pltpu.VMEM((_EB, H2), jnp.float32)],
    )(ea8, Bmat16)


def _den_body(dp_ref, den_ref):
    den_ref[...] = dp_ref[0] + dp_ref[1]


def _den(dpart):
    blk = 1000
    return pl.pallas_call(
        _den_body,
        grid=(N // blk,),
        in_specs=[pl.BlockSpec((NC, blk, H2), lambda i: (0, i, 0))],
        out_specs=pl.BlockSpec((blk, H2), lambda i: (i, 0)),
        out_shape=jax.ShapeDtypeStruct((N, H2), jnp.float32),
    )(dpart)


def _out_body(op_ref, b_ref, o_ref):
    o_ref[...] = op_ref[0] + op_ref[1] + b_ref[...]


def _combine(opart, bias2d):
    blk = 1000
    return pl.pallas_call(
        _out_body,
        grid=(N // blk,),
        in_specs=[pl.BlockSpec((NC, blk, F_IN), lambda i: (0, i, 0)),
                  pl.BlockSpec((1, F_IN), lambda i: (0, 0))],
        out_specs=pl.BlockSpec((blk, F_IN), lambda i: (i, 0)),
        out_shape=jax.ShapeDtypeStruct((N, F_IN), jnp.float32),
    )(opart, bias2d)


def _p1_body(src1, dst2, aer, aloop, asrc, adst, zer16,
             ex, dpart,
             den_sh, src_v, dst_v, asg_v, adg_v, ae_v, ex_v, al_v, sem):
    c = lax.axis_index("c")
    s = lax.axis_index("s")
    wid = c * NS + s
    _copy_rows(s, zer16, den_sh)
    pltpu.sync_copy(aloop, al_v)
    plsc.subcore_barrier()
    alv0 = al_v[0, :]

    def chunk(j, carry):
        off = wid * PER_W + j * B1
        off_ae = pl.multiple_of(jnp.minimum(off, E), 8)
        row0 = wid * (PER_W // HB) + j * NQ1
        pltpu.sync_copy(src1.at[pl.ds(off, B1)], src_v)
        pltpu.sync_copy(dst2.at[pl.ds(row0, NQ1)], dst_v)
        pltpu.sync_copy(aer.at[pl.ds(off_ae, B1)], ae_v)
        cps = []
        for q in range(NQ1):
            cps.append(pltpu.async_copy(
                asrc.at[src_v.at[pl.ds(q * HB, HB)]],
                asg_v.at[pl.ds(q * HB, HB)], sem))
            cps.append(pltpu.async_copy(
                adst.at[dst_v.at[q]], adg_v.at[pl.ds(q * HB, HB)], sem))
        for cp in cps:
            cp.wait()

        def vloop(e, carry2):
            ge = off + e
            a_e = jnp.where(ge < E, ae_v[e, :], alv0)
            a = asg_v[e, :] + adg_v[e, :] + a_e
            a = jnp.where(a >= 0.0, a, SLOPE * a)
            v = jnp.exp(a)
            v = jnp.where(ge < EN, v, 0.0)
            ex_v[e, :] = v
            return carry2

        lax.fori_loop(0, B1, vloop, 0)
        pltpu.sync_copy(ex_v, ex.at[pl.ds(off, B1)])
        for q in range(NQ1):
            pltpu.sync_copy(ex_v.at[pl.ds(q * HB, HB)],
                            den_sh.at[dst_v.at[q]], add=True)
        return carry

    lax.fori_loop(0, NCH1, chunk, 0)
    plsc.subcore_barrier()
    _copy_rows(s, den_sh, dpart.at[c])


_pass1 = functools.partial(
    pl.kernel,
    out_type=[jax.ShapeDtypeStruct((E_PAD, H2), jnp.float32),
              jax.ShapeDtypeStruct((NC, N, H2), jnp.float32)],
    mesh=_mesh,
    scratch_types=[
        pltpu.VMEM_SHARED((N, H2), jnp.float32),
        pltpu.VMEM((B1,), jnp.int32),
        pltpu.VMEM((NQ1, HB), jnp.int32),
        pltpu.VMEM((B1, H2), jnp.float32),
        pltpu.VMEM((B1, H2), jnp.float32),
        pltpu.VMEM((B1, H2), jnp.float32),
        pltpu.VMEM((B1, H2), jnp.float32),
        pltpu.VMEM((8, H2), jnp.float32),
        pltpu.SemaphoreType.DMA,
    ],
    compiler_params=_sc_params,
)(_p1_body)


def _p2_body(src1, dst2, exr, den, h, zer128,
             attn8, opart,
             oacc_sh, src_v, dst_v, ex_v, dg_v, at_v, hr_v, sem_d, sem_h):
    c = lax.axis_index("c")
    s = lax.axis_index("s")
    wid = c * NS + s
    _copy_rows(s, zer128, oacc_sh)
    plsc.subcore_barrier()

    def chunk(j, carry):
        off = wid * PER_W + j * B2
        row0 = wid * (PER_W // HB) + j * NQ2
        pltpu.sync_copy(src1.at[pl.ds(off, B2)], src_v)
        pltpu.sync_copy(dst2.at[pl.ds(row0, NQ2)], dst_v)
        pltpu.sync_copy(exr.at[pl.ds(off, B2)], ex_v)
        dws = []
        hws = []
        for q in range(NQ2):
            dws.append(pltpu.async_copy(
                den.at[dst_v.at[q]], dg_v.at[pl.ds(q * HB, HB)], sem_d))
            hws.append(pltpu.async_copy(
                h.at[src_v.at[pl.ds(q * HB, HB)]],
                hr_v.at[pl.ds(q * HB, HB)], sem_h))
        for g in dws:
            g.wait()

        def vloop(e, carry2):
            at_v[e, :] = ex_v[e, :] / (dg_v[e, :] + 1e-16)
            return carry2

        lax.fori_loop(0, B2, vloop, 0)
        for g in hws:
            g.wait()

        def eloop(e, carry2):
            wv = at_v[e, :]
            for gi in range(H):
                w = wv[gi]
                hv = hr_v[e, pl.ds(gi * 16, 16)]
                hr_v[e, pl.ds(gi * 16, 16)] = hv * w
            return carry2

        lax.fori_loop(0, B2, eloop, 0)
        pltpu.sync_copy(at_v.at[:, pl.ds(0, H)], attn8.at[pl.ds(off, B2)])
        for q in range(NQ2):
            pltpu.sync_copy(hr_v.at[pl.ds(q * HB, HB)],
                            oacc_sh.at[dst_v.at[q]], add=True)
        return carry

    lax.fori_loop(0, NCH2, chunk, 0)
    plsc.subcore_barrier()
    _copy_rows(s, oacc_sh, opart.at[c])


_pass2 = functools.partial(
    pl.kernel,
    out_type=[jax.ShapeDtypeStruct((E_PAD, H), jnp.float32),
              jax.ShapeDtypeStruct((NC, N, F_IN), jnp.float32)],
    mesh=_mesh,
    scratch_types=[
        pltpu.VMEM_SHARED((N, F_IN), jnp.float32),
        pltpu.VMEM((B2,), jnp.int32),
        pltpu.VMEM((NQ2, HB), jnp.int32),
        pltpu.VMEM((B2, H2), jnp.float32),
        pltpu.VMEM((B2, H2), jnp.float32),
        pltpu.VMEM((B2, H2), jnp.float32),
        pltpu.VMEM((B2, F_IN), jnp.float32),
        pltpu.SemaphoreType.DMA,
        pltpu.SemaphoreType.DMA,
    ],
    compiler_params=_sc_params,
)(_p2_body)


def kernel(x, edge_index, edge_attr, W, W_edge, att_src, att_dst, att_edge, bias):
    src = edge_index[0]
    dst = edge_index[1]
    loop = jnp.arange(N, dtype=edge_index.dtype)
    src_f = jnp.concatenate([src, loop])
    dst_f = jnp.concatenate([dst, loop])
    edge_index_full = jnp.stack([src_f, dst_f])

    padlen = E_PAD - EN
    src_p = jnp.concatenate([src_f, jnp.zeros((padlen,), jnp.int32)])
    dst_p = jnp.concatenate(
        [dst_f, jnp.zeros((padlen,), jnp.int32)]).reshape(E_PAD // HB, HB)
    ea8 = jnp.concatenate(
        [jnp.concatenate([edge_attr,
                          jnp.zeros((EA_ROWS - E, D_E), jnp.float32)], axis=0),
         jnp.zeros((EA_ROWS, H - D_E), jnp.float32)], axis=1)

    eye = jnp.repeat(jnp.eye(H, dtype=jnp.float32), C, axis=0)
    Asrc = eye * att_src.reshape(H * C, 1)
    Adst = eye * att_dst.reshape(H * C, 1)
    Asrc16 = jnp.concatenate([Asrc, Asrc], axis=1)
    Adst16 = jnp.concatenate([Adst, Adst], axis=1)
    Bmat = (W_edge.reshape(D_E, H, C) * att_edge[None, :, :]).sum(-1)
    Bmat8 = jnp.concatenate([Bmat, jnp.zeros((H - D_E, H), jnp.float32)], axis=0)
    Bmat16 = jnp.concatenate([Bmat8, Bmat8], axis=1)

    h, asrc, adst = _project(x, W, Asrc16, Adst16)
    aer, aloop = _ae(ea8, Bmat16)

    zer16 = jnp.zeros((N, H2), jnp.float32)
    ex, dpart = _pass1(src_p, dst_p, aer, aloop, asrc, adst, zer16)
    den = _den(dpart)

    zer128 = jnp.zeros((N, F_IN), jnp.float32)
    attn8, opart = _pass2(src_p, dst_p, ex, den, h, zer128)

    out = _combine(opart, bias.reshape(1, F_IN))
    attn = attn8[:EN]
    return out, edge_index_full, attn


# speedup vs baseline: 1.4546x; 1.4546x over previous
import functools

import jax
import jax.numpy as jnp
from jax import lax
from jax.experimental import pallas as pl
from jax.experimental.pallas import tpu as pltpu
from jax.experimental.pallas import tpu_sc as plsc

N = 10000
E = 320000
EN = E + N
F_IN = 128
C = 16
H = 8
H2 = 2 * H
D_E = 4
SLOPE = 0.2

NC = 2
NS = 16
NW = NC * NS
HB = 128
PER_W = 10752
E_PAD = NW * PER_W
B1 = 1536
NQ1 = B1 // HB
NCH1 = PER_W // B1
B2 = 256
NQ2 = B2 // HB
NCH2 = PER_W // B2
EA_ROWS = E + B1
RPS = 640
RPS_LAST = N - 15 * RPS

_mesh = plsc.VectorSubcoreMesh(core_axis_name="c", subcore_axis_name="s")
_sc_params = pltpu.CompilerParams(use_tc_tiling_on_sc=False)


def _copy_rows(s, src, dst):
    @pl.when(s < NS - 1)
    def _():
        off = pl.multiple_of(s * RPS, 8)
        pltpu.sync_copy(src.at[pl.ds(off, RPS)], dst.at[pl.ds(off, RPS)])

    @pl.when(s == NS - 1)
    def _():
        pltpu.sync_copy(src.at[pl.ds(15 * RPS, RPS_LAST)],
                        dst.at[pl.ds(15 * RPS, RPS_LAST)])


def _proj_body(x_ref, w_ref, as_ref, ad_ref, h_ref, asrc_ref, adst_ref):
    h = jnp.dot(x_ref[...], w_ref[...], preferred_element_type=jnp.float32)
    h_ref[...] = h
    asrc_ref[...] = jnp.dot(h, as_ref[...], preferred_element_type=jnp.float32)
    adst_ref[...] = jnp.dot(h, ad_ref[...], preferred_element_type=jnp.float32)


def _project(x, W, Asrc, Adst):
    blk = 1000
    return pl.pallas_call(
        _proj_body,
        grid=(N // blk,),
        in_specs=[pl.BlockSpec((blk, F_IN), lambda i: (i, 0)),
                  pl.BlockSpec((F_IN, F_IN), lambda i: (0, 0)),
                  pl.BlockSpec((F_IN, H2), lambda i: (0, 0)),
                  pl.BlockSpec((F_IN, H2), lambda i: (0, 0))],
        out_specs=[pl.BlockSpec((blk, F_IN), lambda i: (i, 0)),
                   pl.BlockSpec((blk, H2), lambda i: (i, 0)),
                   pl.BlockSpec((blk, H2), lambda i: (i, 0))],
        out_shape=[jax.ShapeDtypeStruct((N, F_IN), jnp.float32),
                   jax.ShapeDtypeStruct((N, H2), jnp.float32),
                   jax.ShapeDtypeStruct((N, H2), jnp.float32)],
    )(x, W, Asrc, Adst)


_EB = EA_ROWS // 64


def _ae_body(ea_ref, bm_ref, ae_ref, al_ref, acc_ref):
    i = pl.program_id(0)
    ae = jnp.dot(ea_ref[...], bm_ref[...], preferred_element_type=jnp.float32)
    ae_ref[...] = ae

    @pl.when(i == 0)
    def _():
        acc_ref[...] = jnp.zeros_like(acc_ref)

    acc_ref[...] += ae

    @pl.when(i == pl.num_programs(0) - 1)
    def _():
        s = jnp.sum(acc_ref[...], axis=0, keepdims=True) * (1.0 / E)
        al_ref[...] = jnp.broadcast_to(s, al_ref.shape)


def _ae(ea4, Bmat16):
    return pl.pallas_call(
        _ae_body,
        grid=(EA_ROWS // _EB,),
        in_specs=[pl.BlockSpec((_EB, D_E), lambda i: (i, 0)),
                  pl.BlockSpec((D_E, H2), lambda i: (0, 0))],
        out_specs=[pl.BlockSpec((_EB, H2), lambda i: (i, 0)),
                   pl.BlockSpec((8, H2), lambda i: (0, 0))],
        out_shape=[jax.ShapeDtypeStruct((EA_ROWS, H2), jnp.float32),
                   jax.ShapeDtypeStruct((8, H2), jnp.float32)],
        scratch_shapes=[pltpu.VMEM((_EB, H2), jnp.float32)],
    )(ea4, Bmat16)


def _den_body(dp_ref, den_ref):
    den_ref[...] = dp_ref[0] + dp_ref[1]


def _den(dpart):
    blk = 1000
    return pl.pallas_call(
        _den_body,
        grid=(N // blk,),
        in_specs=[pl.BlockSpec((NC, blk, H2), lambda i: (0, i, 0))],
        out_specs=pl.BlockSpec((blk, H2), lambda i: (i, 0)),
        out_shape=jax.ShapeDtypeStruct((N, H2), jnp.float32),
    )(dpart)


def _out_body(op_ref, b_ref, o_ref):
    o_ref[...] = op_ref[0] + op_ref[1] + b_ref[...]


def _combine(opart, bias2d):
    blk = 1000
    return pl.pallas_call(
        _out_body,
        grid=(N // blk,),
        in_specs=[pl.BlockSpec((NC, blk, F_IN), lambda i: (0, i, 0)),
                  pl.BlockSpec((1, F_IN), lambda i: (0, 0))],
        out_specs=pl.BlockSpec((blk, F_IN), lambda i: (i, 0)),
        out_shape=jax.ShapeDtypeStruct((N, F_IN), jnp.float32),
    )(opart, bias2d)


def _p1_body(src1, dst1, aer, aloop, asrc, adst, zer16,
             ex, dpart,
             den_sh, src_v, dst_v, asg_v, adg_v, ae_v, ex_v, al_v, sem):
    c = lax.axis_index("c")
    s = lax.axis_index("s")
    wid = c * NS + s
    _copy_rows(s, zer16, den_sh)
    pltpu.sync_copy(aloop, al_v)
    plsc.subcore_barrier()
    alv0 = al_v[0, :]

    def chunk(j, carry):
        off = wid * PER_W + j * B1
        off_ae = pl.multiple_of(jnp.minimum(off, E), 8)
        pltpu.sync_copy(src1.at[pl.ds(off, B1)], src_v)
        for q in range(NQ1):
            pltpu.sync_copy(dst1.at[pl.ds(off + q * HB, HB)], dst_v.at[q])
        pltpu.sync_copy(aer.at[pl.ds(off_ae, B1)], ae_v)
        cps = []
        for q in range(NQ1):
            cps.append(pltpu.async_copy(
                asrc.at[src_v.at[pl.ds(q * HB, HB)]],
                asg_v.at[pl.ds(q * HB, HB)], sem))
            cps.append(pltpu.async_copy(
                adst.at[dst_v.at[q]], adg_v.at[pl.ds(q * HB, HB)], sem))
        for cp in cps:
            cp.wait()

        def vloop(e, carry2):
            ge = off + e
            a_e = jnp.where(ge < E, ae_v[e, :], alv0)
            a = asg_v[e, :] + adg_v[e, :] + a_e
            a = jnp.where(a >= 0.0, a, SLOPE * a)
            v = jnp.exp(a)
            v = jnp.where(ge < EN, v, 0.0)
            ex_v[e, :] = v
            return carry2

        lax.fori_loop(0, B1, vloop, 0)
        pltpu.sync_copy(ex_v, ex.at[pl.ds(off, B1)])
        for q in range(NQ1):
            pltpu.sync_copy(ex_v.at[pl.ds(q * HB, HB)],
                            den_sh.at[dst_v.at[q]], add=True)
        return carry

    lax.fori_loop(0, NCH1, chunk, 0)
    plsc.subcore_barrier()
    _copy_rows(s, den_sh, dpart.at[c])


_pass1 = functools.partial(
    pl.kernel,
    out_type=[jax.ShapeDtypeStruct((E_PAD, H2), jnp.float32),
              jax.ShapeDtypeStruct((NC, N, H2), jnp.float32)],
    mesh=_mesh,
    scratch_types=[
        pltpu.VMEM_SHARED((N, H2), jnp.float32),
        pltpu.VMEM((B1,), jnp.int32),
        pltpu.VMEM((NQ1, HB), jnp.int32),
        pltpu.VMEM((B1, H2), jnp.float32),
        pltpu.VMEM((B1, H2), jnp.float32),
        pltpu.VMEM((B1, H2), jnp.float32),
        pltpu.VMEM((B1, H2), jnp.float32),
        pltpu.VMEM((8, H2), jnp.float32),
        pltpu.SemaphoreType.DMA,
    ],
    compiler_params=_sc_params,
)(_p1_body)


def _p2_body(src1, dst1, exr, den, h, zer128,
             attn8, opart,
             oacc_sh, src_v, dst_v, ex_v, dg_v, at_v, hr_v,
             sem_d, sem_h):
    c = lax.axis_index("c")
    s = lax.axis_index("s")
    wid = c * NS + s
    _copy_rows(s, zer128, oacc_sh)
    plsc.subcore_barrier()

    def chunk(j, carry):
        off = wid * PER_W + j * B2
        pltpu.sync_copy(src1.at[pl.ds(off, B2)], src_v)
        for q in range(NQ2):
            pltpu.sync_copy(dst1.at[pl.ds(off + q * HB, HB)], dst_v.at[q])
        pltpu.sync_copy(exr.at[pl.ds(off, B2)], ex_v)
        dws = []
        hws = []
        for q in range(NQ2):
            dws.append(pltpu.async_copy(
                den.at[dst_v.at[q]], dg_v.at[pl.ds(q * HB, HB)], sem_d))
            hws.append(pltpu.async_copy(
                h.at[src_v.at[pl.ds(q * HB, HB)]],
                hr_v.at[pl.ds(q * HB, HB)], sem_h))
        for g in dws:
            g.wait()

        def vloop(e, carry2):
            at_v[e, :] = ex_v[e, :] / (dg_v[e, :] + 1e-16)
            return carry2

        lax.fori_loop(0, B2, vloop, 0)
        for g in hws:
            g.wait()

        def eloop(e, carry2):
            wv = at_v[e, :]
            for gi in range(H):
                w = wv[gi]
                hv = hr_v[e, pl.ds(gi * 16, 16)]
                hr_v[e, pl.ds(gi * 16, 16)] = hv * w
            return carry2

        lax.fori_loop(0, B2, eloop, 0)
        pltpu.sync_copy(at_v.at[:, pl.ds(0, H)], attn8.at[pl.ds(off, B2)])
        for q in range(NQ2):
            pltpu.sync_copy(hr_v.at[pl.ds(q * HB, HB)],
                            oacc_sh.at[dst_v.at[q]], add=True)
        return carry

    lax.fori_loop(0, NCH2, chunk, 0)
    plsc.subcore_barrier()
    _copy_rows(s, oacc_sh, opart.at[c])


_pass2 = functools.partial(
    pl.kernel,
    out_type=[jax.ShapeDtypeStruct((E_PAD, H), jnp.float32),
              jax.ShapeDtypeStruct((NC, N, F_IN), jnp.float32)],
    mesh=_mesh,
    scratch_types=[
        pltpu.VMEM_SHARED((N, F_IN), jnp.float32),
        pltpu.VMEM((B2,), jnp.int32),
        pltpu.VMEM((NQ2, HB), jnp.int32),
        pltpu.VMEM((B2, H2), jnp.float32),
        pltpu.VMEM((B2, H2), jnp.float32),
        pltpu.VMEM((B2, H2), jnp.float32),
        pltpu.VMEM((B2, F_IN), jnp.float32),
        pltpu.SemaphoreType.DMA,
        pltpu.SemaphoreType.DMA,
    ],
    compiler_params=_sc_params,
)(_p2_body)


def kernel(x, edge_index, edge_attr, W, W_edge, att_src, att_dst, att_edge, bias):
    src = edge_index[0]
    dst = edge_index[1]
    loop = jnp.arange(N, dtype=edge_index.dtype)
    src_f = jnp.concatenate([src, loop])
    dst_f = jnp.concatenate([dst, loop])
    edge_index_full = jnp.stack([src_f, dst_f])

    padlen = E_PAD - EN
    spread = jnp.arange(padlen, dtype=jnp.int32) % N
    src_p = jnp.concatenate([src_f, spread])
    dst_p = jnp.concatenate([dst_f, spread])
    ea4 = jnp.concatenate(
        [edge_attr, jnp.zeros((EA_ROWS - E, D_E), jnp.float32)], axis=0)

    eye = jnp.repeat(jnp.eye(H, dtype=jnp.float32), C, axis=0)
    Asrc = eye * att_src.reshape(H * C, 1)
    Adst = eye * att_dst.reshape(H * C, 1)
    Asrc16 = jnp.concatenate([Asrc, Asrc], axis=1)
    Adst16 = jnp.concatenate([Adst, Adst], axis=1)
    Bmat = (W_edge.reshape(D_E, H, C) * att_edge[None, :, :]).sum(-1)
    Bmat16 = jnp.concatenate([Bmat, Bmat], axis=1)

    h, asrc, adst = _project(x, W, Asrc16, Adst16)
    aer, aloop = _ae(ea4, Bmat16)

    zer16 = jnp.zeros((N, H2), jnp.float32)
    ex, dpart = _pass1(src_p, dst_p, aer, aloop, asrc, adst, zer16)
    den = _den(dpart)

    zer128 = jnp.zeros((N, F_IN), jnp.float32)
    attn8, opart = _pass2(src_p, dst_p, ex, den, h, zer128)

    out = _combine(opart, bias.reshape(1, F_IN))
    attn = attn8[:EN]
    return out, edge_index_full, attn


# speedup vs baseline: 1.4597x; 1.0035x over previous
import functools

import jax
import jax.numpy as jnp
from jax import lax
from jax.experimental import pallas as pl
from jax.experimental.pallas import tpu as pltpu
from jax.experimental.pallas import tpu_sc as plsc

N = 10000
E = 320000
EN = E + N
F_IN = 128
C = 16
H = 8
H2 = 2 * H
D_E = 4
SLOPE = 0.2

NC = 2
NS = 16
NW = NC * NS
HB = 128
PER_W = 10752
E_PAD = NW * PER_W
B1 = 1536
NQ1 = B1 // HB
NCH1 = PER_W // B1
B2 = 128
NCH2 = PER_W // B2
EA_ROWS = E + B1
RPS = 640
RPS_LAST = N - 15 * RPS

_mesh = plsc.VectorSubcoreMesh(core_axis_name="c", subcore_axis_name="s")
_sc_params = pltpu.CompilerParams(use_tc_tiling_on_sc=False)


def _copy_rows(s, src, dst):
    @pl.when(s < NS - 1)
    def _():
        off = pl.multiple_of(s * RPS, 8)
        pltpu.sync_copy(src.at[pl.ds(off, RPS)], dst.at[pl.ds(off, RPS)])

    @pl.when(s == NS - 1)
    def _():
        pltpu.sync_copy(src.at[pl.ds(15 * RPS, RPS_LAST)],
                        dst.at[pl.ds(15 * RPS, RPS_LAST)])


def _proj_body(x_ref, w_ref, as_ref, ad_ref, h_ref, asrc_ref, adst_ref):
    h = jnp.dot(x_ref[...], w_ref[...], preferred_element_type=jnp.float32)
    h_ref[...] = h
    asrc_ref[...] = jnp.dot(h, as_ref[...], preferred_element_type=jnp.float32)
    adst_ref[...] = jnp.dot(h, ad_ref[...], preferred_element_type=jnp.float32)


def _project(x, W, Asrc, Adst):
    blk = 1000
    return pl.pallas_call(
        _proj_body,
        grid=(N // blk,),
        in_specs=[pl.BlockSpec((blk, F_IN), lambda i: (i, 0)),
                  pl.BlockSpec((F_IN, F_IN), lambda i: (0, 0)),
                  pl.BlockSpec((F_IN, H2), lambda i: (0, 0)),
                  pl.BlockSpec((F_IN, H2), lambda i: (0, 0))],
        out_specs=[pl.BlockSpec((blk, F_IN), lambda i: (i, 0)),
                   pl.BlockSpec((blk, H2), lambda i: (i, 0)),
                   pl.BlockSpec((blk, H2), lambda i: (i, 0))],
        out_shape=[jax.ShapeDtypeStruct((N, F_IN), jnp.float32),
                   jax.ShapeDtypeStruct((N, H2), jnp.float32),
                   jax.ShapeDtypeStruct((N, H2), jnp.float32)],
    )(x, W, Asrc, Adst)


_EB = EA_ROWS // 64


def _ae_body(ea_ref, bm_ref, ae_ref, al_ref, acc_ref):
    i = pl.program_id(0)
    ae = jnp.dot(ea_ref[...], bm_ref[...], preferred_element_type=jnp.float32)
    ae_ref[...] = ae

    @pl.when(i == 0)
    def _():
        acc_ref[...] = jnp.zeros_like(acc_ref)

    acc_ref[...] += ae

    @pl.when(i == pl.num_programs(0) - 1)
    def _():
        s = jnp.sum(acc_ref[...], axis=0, keepdims=True) * (1.0 / E)
        al_ref[...] = jnp.broadcast_to(s, al_ref.shape)


def _ae(ea4, Bmat16):
    return pl.pallas_call(
        _ae_body,
        grid=(EA_ROWS // _EB,),
        in_specs=[pl.BlockSpec((_EB, D_E), lambda i: (i, 0)),
                  pl.BlockSpec((D_E, H2), lambda i: (0, 0))],
        out_specs=[pl.BlockSpec((_EB, H2), lambda i: (i, 0)),
                   pl.BlockSpec((8, H2), lambda i: (0, 0))],
        out_shape=[jax.ShapeDtypeStruct((EA_ROWS, H2), jnp.float32),
                   jax.ShapeDtypeStruct((8, H2), jnp.float32)],
        scratch_shapes=[pltpu.VMEM((_EB, H2), jnp.float32)],
    )(ea4, Bmat16)


def _den_body(dp_ref, den_ref):
    den_ref[...] = dp_ref[0] + dp_ref[1]


def _den(dpart):
    blk = 1000
    return pl.pallas_call(
        _den_body,
        grid=(N // blk,),
        in_specs=[pl.BlockSpec((NC, blk, H2), lambda i: (0, i, 0))],
        out_specs=pl.BlockSpec((blk, H2), lambda i: (i, 0)),
        out_shape=jax.ShapeDtypeStruct((N, H2), jnp.float32),
    )(dpart)


def _out_body(op_ref, b_ref, o_ref):
    o_ref[...] = op_ref[0] + op_ref[1] + b_ref[...]


def _combine(opart, bias2d):
    blk = 1000
    return pl.pallas_call(
        _out_body,
        grid=(N // blk,),
        in_specs=[pl.BlockSpec((NC, blk, F_IN), lambda i: (0, i, 0)),
                  pl.BlockSpec((1, F_IN), lambda i: (0, 0))],
        out_specs=pl.BlockSpec((blk, F_IN), lambda i: (i, 0)),
        out_shape=jax.ShapeDtypeStruct((N, F_IN), jnp.float32),
    )(opart, bias2d)


def _p1_body(src1, dst1, aer, aloop, asrc, adst, zer16,
             ex, dpart,
             den_sh, src_v, dst_v, asg_v, adg_v, ae_v, ex_v, al_v, sem):
    c = lax.axis_index("c")
    s = lax.axis_index("s")
    wid = c * NS + s
    _copy_rows(s, zer16, den_sh)
    pltpu.sync_copy(aloop, al_v)
    plsc.subcore_barrier()
    alv0 = al_v[0, :]

    def chunk(j, carry):
        off = wid * PER_W + j * B1
        off_ae = pl.multiple_of(jnp.minimum(off, E), 8)
        pltpu.sync_copy(src1.at[pl.ds(off, B1)], src_v)
        for q in range(NQ1):
            pltpu.sync_copy(dst1.at[pl.ds(off + q * HB, HB)], dst_v.at[q])
        pltpu.sync_copy(aer.at[pl.ds(off_ae, B1)], ae_v)
        cps = []
        for q in range(NQ1):
            cps.append(pltpu.async_copy(
                asrc.at[src_v.at[pl.ds(q * HB, HB)]],
                asg_v.at[pl.ds(q * HB, HB)], sem))
            cps.append(pltpu.async_copy(
                adst.at[dst_v.at[q]], adg_v.at[pl.ds(q * HB, HB)], sem))
        for cp in cps:
            cp.wait()

        def vloop(e, carry2):
            ge = off + e
            a_e = jnp.where(ge < E, ae_v[e, :], alv0)
            a = asg_v[e, :] + adg_v[e, :] + a_e
            a = jnp.where(a >= 0.0, a, SLOPE * a)
            v = jnp.exp(a)
            v = jnp.where(ge < EN, v, 0.0)
            ex_v[e, :] = v
            return carry2

        lax.fori_loop(0, B1, vloop, 0)
        pltpu.sync_copy(ex_v, ex.at[pl.ds(off, B1)])
        for q in range(NQ1):
            pltpu.sync_copy(ex_v.at[pl.ds(q * HB, HB)],
                            den_sh.at[dst_v.at[q]], add=True)
        return carry

    lax.fori_loop(0, NCH1, chunk, 0)
    plsc.subcore_barrier()
    _copy_rows(s, den_sh, dpart.at[c])


_pass1 = functools.partial(
    pl.kernel,
    out_type=[jax.ShapeDtypeStruct((E_PAD, H2), jnp.float32),
              jax.ShapeDtypeStruct((NC, N, H2), jnp.float32)],
    mesh=_mesh,
    scratch_types=[
        pltpu.VMEM_SHARED((N, H2), jnp.float32),
        pltpu.VMEM((B1,), jnp.int32),
        pltpu.VMEM((NQ1, HB), jnp.int32),
        pltpu.VMEM((B1, H2), jnp.float32),
        pltpu.VMEM((B1, H2), jnp.float32),
        pltpu.VMEM((B1, H2), jnp.float32),
        pltpu.VMEM((B1, H2), jnp.float32),
        pltpu.VMEM((8, H2), jnp.float32),
        pltpu.SemaphoreType.DMA,
    ],
    compiler_params=_sc_params,
)(_p1_body)


def _p2_body(src1, dst1, exr, den, h, zer128,
             attn8, opart,
             oacc_sh,
             src_v0, dst_v0, ex_v0, dg_v0, at_v0, hr_v0,
             src_v1, dst_v1, ex_v1, dg_v1, at_v1, hr_v1,
             sem_d0, sem_h0, sem_d1, sem_h1):
    c = lax.axis_index("c")
    s = lax.axis_index("s")
    wid = c * NS + s
    base = wid * PER_W
    _copy_rows(s, zer128, oacc_sh)
    plsc.subcore_barrier()

    bufs = [(src_v0, dst_v0, ex_v0, dg_v0, at_v0, hr_v0, sem_d0, sem_h0),
            (src_v1, dst_v1, ex_v1, dg_v1, at_v1, hr_v1, sem_d1, sem_h1)]

    def fire(j, p):
        src_v, dst_v, ex_v, dg_v, at_v, hr_v, sem_d, sem_h = bufs[p]
        off = base + j * B2
        pltpu.sync_copy(src1.at[pl.ds(off, B2)], src_v)
        pltpu.sync_copy(dst1.at[pl.ds(off, B2)], dst_v.at[0])
        pltpu.sync_copy(exr.at[pl.ds(off, B2)], ex_v)
        pltpu.async_copy(den.at[dst_v.at[0]], dg_v, sem_d)
        pltpu.async_copy(h.at[src_v], hr_v, sem_h)

    def compute(j, p):
        src_v, dst_v, ex_v, dg_v, at_v, hr_v, sem_d, sem_h = bufs[p]
        off = base + j * B2
        pltpu.make_async_copy(den.at[dst_v.at[0]], dg_v, sem_d).wait()

        def vloop(e, carry2):
            at_v[e, :] = ex_v[e, :] / (dg_v[e, :] + 1e-16)
            return carry2

        lax.fori_loop(0, B2, vloop, 0)
        pltpu.make_async_copy(h.at[src_v], hr_v, sem_h).wait()

        def eloop(e, carry2):
            wv = at_v[e, :]
            for gi in range(H):
                w = wv[gi]
                hv = hr_v[e, pl.ds(gi * 16, 16)]
                hr_v[e, pl.ds(gi * 16, 16)] = hv * w
            return carry2

        lax.fori_loop(0, B2, eloop, 0)
        pltpu.sync_copy(at_v.at[:, pl.ds(0, H)], attn8.at[pl.ds(off, B2)])
        pltpu.sync_copy(hr_v, oacc_sh.at[dst_v.at[0]], add=True)

    fire(0, 0)

    def pair(t, carry):
        j0 = t * 2
        fire(j0 + 1, 1)
        compute(j0, 0)

        @pl.when(j0 + 2 < NCH2)
        def _():
            fire(j0 + 2, 0)

        compute(j0 + 1, 1)
        return carry

    lax.fori_loop(0, NCH2 // 2, pair, 0)
    plsc.subcore_barrier()
    _copy_rows(s, oacc_sh, opart.at[c])


def _p2_scr():
    return [
        pltpu.VMEM((B2,), jnp.int32),
        pltpu.VMEM((1, B2), jnp.int32),
        pltpu.VMEM((B2, H2), jnp.float32),
        pltpu.VMEM((B2, H2), jnp.float32),
        pltpu.VMEM((B2, H2), jnp.float32),
        pltpu.VMEM((B2, F_IN), jnp.float32),
    ]


_pass2 = functools.partial(
    pl.kernel,
    out_type=[jax.ShapeDtypeStruct((E_PAD, H), jnp.float32),
              jax.ShapeDtypeStruct((NC, N, F_IN), jnp.float32)],
    mesh=_mesh,
    scratch_types=(
        [pltpu.VMEM_SHARED((N, F_IN), jnp.float32)]
        + _p2_scr() + _p2_scr()
        + [pltpu.SemaphoreType.DMA] * 4
    ),
    compiler_params=_sc_params,
)(_p2_body)


def kernel(x, edge_index, edge_attr, W, W_edge, att_src, att_dst, att_edge, bias):
    src = edge_index[0]
    dst = edge_index[1]
    loop = jnp.arange(N, dtype=edge_index.dtype)
    src_f = jnp.concatenate([src, loop])
    dst_f = jnp.concatenate([dst, loop])
    edge_index_full = jnp.stack([src_f, dst_f])

    padlen = E_PAD - EN
    spread = jnp.arange(padlen, dtype=jnp.int32) % N
    src_p = jnp.concatenate([src_f, spread])
    dst_p = jnp.concatenate([dst_f, spread])
    ea4 = jnp.concatenate(
        [edge_attr, jnp.zeros((EA_ROWS - E, D_E), jnp.float32)], axis=0)

    eye = jnp.repeat(jnp.eye(H, dtype=jnp.float32), C, axis=0)
    Asrc = eye * att_src.reshape(H * C, 1)
    Adst = eye * att_dst.reshape(H * C, 1)
    Asrc16 = jnp.concatenate([Asrc, Asrc], axis=1)
    Adst16 = jnp.concatenate([Adst, Adst], axis=1)
    Bmat = (W_edge.reshape(D_E, H, C) * att_edge[None, :, :]).sum(-1)
    Bmat16 = jnp.concatenate([Bmat, Bmat], axis=1)

    h, asrc, adst = _project(x, W, Asrc16, Adst16)
    aer, aloop = _ae(ea4, Bmat16)

    zer16 = jnp.zeros((N, H2), jnp.float32)
    ex, dpart = _pass1(src_p, dst_p, aer, aloop, asrc, adst, zer16)
    den = _den(dpart)

    zer128 = jnp.zeros((N, F_IN), jnp.float32)
    attn8, opart = _pass2(src_p, dst_p, ex, den, h, zer128)

    out = _combine(opart, bias.reshape(1, F_IN))
    attn = attn8[:EN]
    return out, edge_index_full, attn
